# baseline (speedup 1.0000x reference)
import functools

import jax
import jax.numpy as jnp
from jax import lax
from jax.experimental import pallas as pl
from jax.experimental.pallas import tpu as pltpu
from jax.experimental.pallas import tpu_sc as plsc

NEG_SLOPE = 0.2
PAD = -1e30
EPS = 1e-9


def _dense_body(x_ref, w_ref, b_ref, o_ref):
    o_ref[...] = (
        jnp.dot(x_ref[...], w_ref[...], preferred_element_type=jnp.float32)
        + b_ref[...]
    )


def _dense(x, w, b, br=1000):
    n, d = x.shape
    wout = w.shape[1]
    return pl.pallas_call(
        _dense_body,
        grid=(n // br,),
        in_specs=[
            pl.BlockSpec((br, d), lambda i: (i, 0)),
            pl.BlockSpec((d, wout), lambda i: (0, 0)),
            pl.BlockSpec((1, wout), lambda i: (0, 0)),
        ],
        out_specs=pl.BlockSpec((br, wout), lambda i: (i, 0)),
        out_shape=jax.ShapeDtypeStruct((n, wout), jnp.float32),
    )(x, w, b)


def _mid_body(cp_ref, eexp_ref, ws2_ref, wd2_ref, s2_ref, d2_ref, *, wm):
    u = cp_ref[0, :, :wm] + cp_ref[1, :, :wm]
    d = cp_ref[0, :, wm:] + cp_ref[1, :, wm:]
    dexp = jnp.dot(d, eexp_ref[...], preferred_element_type=jnp.float32)
    z = u / (dexp + EPS)
    h = jnp.where(z > 0, z, jnp.exp(z) - 1.0)
    s2_ref[...] = jnp.dot(h, ws2_ref[...], preferred_element_type=jnp.float32)
    d2_ref[...] = jnp.dot(h, wd2_ref[...], preferred_element_type=jnp.float32)


def _mid(comb, eexp, ws2, wd2, br=1000):
    n = comb.shape[1]
    ws = comb.shape[2]
    wm = ws - 16
    w2s = ws2.shape[1]
    w2d = wd2.shape[1]
    return pl.pallas_call(
        functools.partial(_mid_body, wm=wm),
        grid=(n // br,),
        in_specs=[
            pl.BlockSpec((2, br, ws), lambda i: (0, i, 0)),
            pl.BlockSpec((16, wm), lambda i: (0, 0)),
            pl.BlockSpec((wm, w2s), lambda i: (0, 0)),
            pl.BlockSpec((wm, w2d), lambda i: (0, 0)),
        ],
        out_specs=(
            pl.BlockSpec((br, w2s), lambda i: (i, 0)),
            pl.BlockSpec((br, w2d), lambda i: (i, 0)),
        ),
        out_shape=(
            jax.ShapeDtypeStruct((n, w2s), jnp.float32),
            jax.ShapeDtypeStruct((n, w2d), jnp.float32),
        ),
    )(comb, eexp, ws2, wd2)


def _final_body(cp_ref, o_ref, *, wm):
    u = cp_ref[0, :, :wm] + cp_ref[1, :, :wm]
    d = cp_ref[0, :, wm:] + cp_ref[1, :, wm:]
    o_ref[...] = u / (d + EPS)


def _final(comb, br=1000):
    n = comb.shape[1]
    ws = comb.shape[2]
    wm = ws - 16
    return pl.pallas_call(
        functools.partial(_final_body, wm=wm),
        grid=(n // br,),
        in_specs=[
            pl.BlockSpec((2, br, ws), lambda i: (0, i, 0)),
        ],
        out_specs=pl.BlockSpec((br, wm), lambda i: (i, 0)),
        out_shape=jax.ShapeDtypeStruct((n, wm), jnp.float32),
    )(comb)


def _edge_pass(srctab, dsttab, sidx, didx, z, *, wm, expand_heads):
    n = srctab.shape[0]
    e = sidx.shape[0]
    ws = wm + 16
    nc, ns = 2, 16
    nw = nc * ns
    ep = e // nw
    k_ch = min(200 if ws > 32 else 500, ep)
    nch = ep // k_ch
    nt = nch // 3
    rem = nch % 3
    rowb = 400
    nrb = n // rowb
    nf = wm // 16

    sidx3 = sidx.reshape(nw, nch, k_ch)
    didx3 = didx.reshape(nw, nch, k_ch)

    mesh = plsc.VectorSubcoreMesh(
        core_axis_name="c", subcore_axis_name="s",
        num_cores=nc, num_subcores=ns)

    @functools.partial(
        pl.kernel,
        compiler_params=pltpu.CompilerParams(
            use_tc_tiling_on_sc=False, needs_layout_passes=False),
        out_type=jax.ShapeDtypeStruct((nc, n, ws), jnp.float32),
        mesh=mesh,
        scratch_types=[
            pltpu.VMEM((nch, k_ch), jnp.int32),
            pltpu.VMEM((nch, k_ch), jnp.int32),
            pltpu.VMEM((k_ch, ws), jnp.float32),
            pltpu.VMEM((k_ch, ws), jnp.float32),
            pltpu.VMEM((k_ch, ws), jnp.float32),
            pltpu.VMEM((k_ch, 16), jnp.float32),
            pltpu.VMEM((k_ch, 16), jnp.float32),
            pltpu.VMEM((k_ch, 16), jnp.float32),
            pltpu.VMEM_SHARED((n, ws), jnp.float32),
            pltpu.SemaphoreType.DMA,
            pltpu.SemaphoreType.DMA,
            pltpu.SemaphoreType.DMA,
            pltpu.SemaphoreType.DMA,
            pltpu.SemaphoreType.DMA,
            pltpu.SemaphoreType.DMA,
            pltpu.SemaphoreType.DMA,
            pltpu.SemaphoreType.DMA,
            pltpu.SemaphoreType.DMA,
        ],
    )
    def kfn(srct_h, dstt_h, sidx_h, didx_h, z_h, comb_h,
            sidx_b, didx_b, srows0, srows1, srows2, drows0, drows1, drows2,
            acc, sem_s0, sem_s1, sem_s2, sem_d0, sem_d1, sem_d2,
            sem_c0, sem_c1, sem_c2):
        cid = lax.axis_index("c")
        sid = lax.axis_index("s")
        wid = sid * nc + cid
        srows = (srows0, srows1, srows2)
        drows = (drows0, drows1, drows2)
        sem_s = (sem_s0, sem_s1, sem_s2)
        sem_d = (sem_d0, sem_d1, sem_d2)
        sem_c = (sem_c0, sem_c1, sem_c2)

        def row_blocks(fn):
            for it in range((nrb + ns - 1) // ns):
                rb = sid + it * ns
                if (it + 1) * ns <= nrb:
                    fn(pl.multiple_of(rb * rowb, 8))
                else:
                    @pl.when(rb < nrb)
                    def _():
                        fn(pl.multiple_of(rb * rowb, 8))

        def init_rows(r0):
            pltpu.sync_copy(z_h.at[pl.ds(r0, rowb)], acc.at[pl.ds(r0, rowb)])

        pltpu.sync_copy(sidx_h.at[wid], sidx_b)
        pltpu.sync_copy(didx_h.at[wid], didx_b)
        row_blocks(init_rows)
        plsc.subcore_barrier()
        div8 = lax.iota(jnp.int32, 16) >> 3

        def gfire(cc, b):
            pltpu.async_copy(srct_h.at[sidx_b.at[cc]], srows[b], sem_s[b])
            pltpu.async_copy(dstt_h.at[didx_b.at[cc]], drows[b], sem_d[b])

        def gwait(b):
            pltpu.make_async_copy(
                srct_h.at[sidx_b.at[0]], srows[b], sem_s[b]).wait()
            pltpu.make_async_copy(
                dstt_h.at[didx_b.at[0]], drows[b], sem_d[b]).wait()

        def sfire(cc, b):
            pltpu.async_copy(
                srows[b], acc.at[didx_b.at[cc]], sem_c[b], add=True)

        def swait(b):
            pltpu.make_async_copy(
                srows[b], acc.at[didx_b.at[0]], sem_c[b]).wait()

        def compute(b):
            sr = srows[b]
            dr = drows[b]

            def edge(k, carry2):
                el = sr[k, pl.ds(wm, 16)]
                er = dr[k, :]
                t = el + er
                s = jnp.exp(jnp.maximum(t, NEG_SLOPE * t))
                sr[k, pl.ds(wm, 16)] = s
                for j in range(nf):
                    if expand_heads:
                        ex = jnp.take_along_axis(s, div8 + (2 * j), axis=0)
                    else:
                        ex = s
                    sr[k, pl.ds(16 * j, 16)] = (
                        sr[k, pl.ds(16 * j, 16)] * ex
                    )
                return carry2

            lax.fori_loop(0, k_ch, edge, 0, unroll=4)

        def do_chunk(cc, u, maybe_first):
            b = u % 3
            pb = (u + 2) % 3
            gwait(b)
            compute(b)
            if maybe_first:
                @pl.when(cc >= 1)
                def _():
                    swait(pb)
            else:
                swait(pb)
            if isinstance(cc, int):
                if cc + 2 < nch:
                    gfire(cc + 2, pb)
            else:
                @pl.when(cc + 2 < nch)
                def _():
                    gfire(jnp.minimum(cc + 2, nch - 1), pb)
            sfire(cc, b)

        gfire(0, 0)
        if nch > 1:
            gfire(1, 1)

        def triple(i, carry):
            c0 = 3 * i
            do_chunk(c0, 0, True)
            do_chunk(c0 + 1, 1, False)
            do_chunk(c0 + 2, 2, False)
            return carry

        lax.fori_loop(0, nt, triple, 0)
        for r in range(rem):
            do_chunk(3 * nt + r, r, 3 * nt + r == 0)
        swait((nch - 1) % 3)
        plsc.subcore_barrier()

        def write_rows(r0):
            pltpu.sync_copy(acc.at[pl.ds(r0, rowb)],
                            comb_h.at[cid, pl.ds(r0, rowb)])

        row_blocks(write_rows)

    return kfn(srctab, dsttab, sidx3, didx3, z)


def kernel(x, edge_index, W1, al1, ar1, W2, al2, ar2):
    n, d_in = x.shape
    h1, f1 = al1.shape
    h2, f2 = al2.shape
    wm1 = h1 * f1
    wm2 = h2 * f2

    idx = jnp.arange(wm1)
    a_l1 = jnp.zeros((wm1, h1), jnp.float32).at[idx, idx // f1].set(
        al1.reshape(wm1))
    a_r1 = jnp.zeros((wm1, h1), jnp.float32).at[idx, idx // f1].set(
        ar1.reshape(wm1))
    ws1 = jnp.concatenate(
        [W1, W1 @ a_l1, jnp.zeros((d_in, 16 - h1), jnp.float32)], axis=1)
    bs1 = jnp.concatenate(
        [jnp.zeros((wm1 + h1,), jnp.float32),
         jnp.full((16 - h1,), PAD, jnp.float32)])[None, :]
    wd1 = jnp.concatenate(
        [W1 @ a_r1, jnp.zeros((d_in, 16 - h1), jnp.float32)], axis=1)
    bd1 = jnp.concatenate(
        [jnp.zeros((h1,), jnp.float32),
         jnp.full((16 - h1,), PAD, jnp.float32)])[None, :]

    eexp = jnp.zeros((16, wm1), jnp.float32).at[idx // f1, idx].set(1.0)

    w_el2 = W2 @ al2.reshape(wm2)
    w_er2 = W2 @ ar2.reshape(wm2)
    ws2 = jnp.concatenate([W2, jnp.tile(w_el2[:, None], (1, 16))], axis=1)
    wd2 = jnp.tile(w_er2[:, None], (1, 16))

    sidx = edge_index[0]
    didx = edge_index[1]
    z1 = jnp.zeros((n, wm1 + 16), jnp.float32)
    z2 = jnp.zeros((n, wm2 + 16), jnp.float32)

    srctab1 = _dense(x, ws1, bs1)
    dsttab1 = _dense(x, wd1, bd1)
    comb1 = _edge_pass(srctab1, dsttab1, sidx, didx, z1,
                       wm=wm1, expand_heads=True)

    srctab2, dsttab2 = _mid(comb1, eexp, ws2, wd2)

    comb2 = _edge_pass(srctab2, dsttab2, sidx, didx, z2,
                       wm=wm2, expand_heads=False)

    return _final(comb2)


# speedup vs baseline: 1.0056x; 1.0056x over previous
import functools

import jax
import jax.numpy as jnp
from jax import lax
from jax.experimental import pallas as pl
from jax.experimental.pallas import tpu as pltpu
from jax.experimental.pallas import tpu_sc as plsc

NEG_SLOPE = 0.2
PAD = -1e30
EPS = 1e-9


def _dense_body(x_ref, w_ref, b_ref, o_ref):
    o_ref[...] = (
        jnp.dot(x_ref[...], w_ref[...], preferred_element_type=jnp.float32)
        + b_ref[...]
    )


def _dense(x, w, b, br=1000):
    n, d = x.shape
    wout = w.shape[1]
    return pl.pallas_call(
        _dense_body,
        grid=(n // br,),
        in_specs=[
            pl.BlockSpec((br, d), lambda i: (i, 0)),
            pl.BlockSpec((d, wout), lambda i: (0, 0)),
            pl.BlockSpec((1, wout), lambda i: (0, 0)),
        ],
        out_specs=pl.BlockSpec((br, wout), lambda i: (i, 0)),
        out_shape=jax.ShapeDtypeStruct((n, wout), jnp.float32),
    )(x, w, b)


def _mid_body(cp_ref, eexp_ref, ws2_ref, wd2_ref, s2_ref, d2_ref, *, wm):
    u = cp_ref[0, :, :wm] + cp_ref[1, :, :wm]
    d = cp_ref[0, :, wm:] + cp_ref[1, :, wm:]
    dexp = jnp.dot(d, eexp_ref[...], preferred_element_type=jnp.float32)
    z = u / (dexp + EPS)
    h = jnp.where(z > 0, z, jnp.exp(z) - 1.0)
    s2_ref[...] = jnp.dot(h, ws2_ref[...], preferred_element_type=jnp.float32)
    d2_ref[...] = jnp.dot(h, wd2_ref[...], preferred_element_type=jnp.float32)


def _mid(comb, eexp, ws2, wd2, br=1000):
    n = comb.shape[1]
    ws = comb.shape[2]
    wm = ws - 16
    w2s = ws2.shape[1]
    w2d = wd2.shape[1]
    return pl.pallas_call(
        functools.partial(_mid_body, wm=wm),
        grid=(n // br,),
        in_specs=[
            pl.BlockSpec((2, br, ws), lambda i: (0, i, 0)),
            pl.BlockSpec((16, wm), lambda i: (0, 0)),
            pl.BlockSpec((wm, w2s), lambda i: (0, 0)),
            pl.BlockSpec((wm, w2d), lambda i: (0, 0)),
        ],
        out_specs=(
            pl.BlockSpec((br, w2s), lambda i: (i, 0)),
            pl.BlockSpec((br, w2d), lambda i: (i, 0)),
        ),
        out_shape=(
            jax.ShapeDtypeStruct((n, w2s), jnp.float32),
            jax.ShapeDtypeStruct((n, w2d), jnp.float32),
        ),
    )(comb, eexp, ws2, wd2)


def _final_body(cp_ref, o_ref, *, wm):
    u = cp_ref[0, :, :wm] + cp_ref[1, :, :wm]
    d = cp_ref[0, :, wm:] + cp_ref[1, :, wm:]
    o_ref[...] = u / (d + EPS)


def _final(comb, br=1000):
    n = comb.shape[1]
    ws = comb.shape[2]
    wm = ws - 16
    return pl.pallas_call(
        functools.partial(_final_body, wm=wm),
        grid=(n // br,),
        in_specs=[
            pl.BlockSpec((2, br, ws), lambda i: (0, i, 0)),
        ],
        out_specs=pl.BlockSpec((br, wm), lambda i: (i, 0)),
        out_shape=jax.ShapeDtypeStruct((n, wm), jnp.float32),
    )(comb)


def _edge_pass(srctab, dsttab, sidx, didx, z, *, wm, expand_heads):
    n = srctab.shape[0]
    e = sidx.shape[0]
    ws = wm + 16
    nc, ns = 2, 16
    nw = nc * ns
    ep = e // nw
    k_ch = min(200, ep)
    nch = ep // k_ch
    nt = nch // 3
    rem = nch % 3
    rowb = 400
    nrb = n // rowb
    nf = wm // 16

    sidx3 = sidx.reshape(nw, nch, k_ch)
    didx3 = didx.reshape(nw, nch, k_ch)

    mesh = plsc.VectorSubcoreMesh(
        core_axis_name="c", subcore_axis_name="s",
        num_cores=nc, num_subcores=ns)

    @functools.partial(
        pl.kernel,
        compiler_params=pltpu.CompilerParams(
            use_tc_tiling_on_sc=False, needs_layout_passes=False),
        out_type=jax.ShapeDtypeStruct((nc, n, ws), jnp.float32),
        mesh=mesh,
        scratch_types=[
            pltpu.VMEM((nch, k_ch), jnp.int32),
            pltpu.VMEM((nch, k_ch), jnp.int32),
            pltpu.VMEM((k_ch, ws), jnp.float32),
            pltpu.VMEM((k_ch, ws), jnp.float32),
            pltpu.VMEM((k_ch, ws), jnp.float32),
            pltpu.VMEM((k_ch, 16), jnp.float32),
            pltpu.VMEM((k_ch, 16), jnp.float32),
            pltpu.VMEM((k_ch, 16), jnp.float32),
            pltpu.VMEM_SHARED((n, ws), jnp.float32),
            pltpu.SemaphoreType.DMA,
            pltpu.SemaphoreType.DMA,
            pltpu.SemaphoreType.DMA,
            pltpu.SemaphoreType.DMA,
            pltpu.SemaphoreType.DMA,
            pltpu.SemaphoreType.DMA,
            pltpu.SemaphoreType.DMA,
            pltpu.SemaphoreType.DMA,
            pltpu.SemaphoreType.DMA,
        ],
    )
    def kfn(srct_h, dstt_h, sidx_h, didx_h, z_h, comb_h,
            sidx_b, didx_b, srows0, srows1, srows2, drows0, drows1, drows2,
            acc, sem_s0, sem_s1, sem_s2, sem_d0, sem_d1, sem_d2,
            sem_c0, sem_c1, sem_c2):
        cid = lax.axis_index("c")
        sid = lax.axis_index("s")
        wid = sid * nc + cid
        srows = (srows0, srows1, srows2)
        drows = (drows0, drows1, drows2)
        sem_s = (sem_s0, sem_s1, sem_s2)
        sem_d = (sem_d0, sem_d1, sem_d2)
        sem_c = (sem_c0, sem_c1, sem_c2)

        def row_blocks(fn):
            for it in range((nrb + ns - 1) // ns):
                rb = sid + it * ns
                if (it + 1) * ns <= nrb:
                    fn(pl.multiple_of(rb * rowb, 8))
                else:
                    @pl.when(rb < nrb)
                    def _():
                        fn(pl.multiple_of(rb * rowb, 8))

        def init_rows(r0):
            pltpu.sync_copy(z_h.at[pl.ds(r0, rowb)], acc.at[pl.ds(r0, rowb)])

        pltpu.sync_copy(sidx_h.at[wid], sidx_b)
        pltpu.sync_copy(didx_h.at[wid], didx_b)
        row_blocks(init_rows)
        plsc.subcore_barrier()
        div8 = lax.iota(jnp.int32, 16) >> 3

        def gfire(cc, b):
            pltpu.async_copy(srct_h.at[sidx_b.at[cc]], srows[b], sem_s[b])
            pltpu.async_copy(dstt_h.at[didx_b.at[cc]], drows[b], sem_d[b])

        def gwait(b):
            pltpu.make_async_copy(
                srct_h.at[sidx_b.at[0]], srows[b], sem_s[b]).wait()
            pltpu.make_async_copy(
                dstt_h.at[didx_b.at[0]], drows[b], sem_d[b]).wait()

        def sfire(cc, b):
            pltpu.async_copy(
                srows[b], acc.at[didx_b.at[cc]], sem_c[b], add=True)

        def swait(b):
            pltpu.make_async_copy(
                srows[b], acc.at[didx_b.at[0]], sem_c[b]).wait()

        def compute(b):
            sr = srows[b]
            dr = drows[b]

            def edge(k, carry2):
                el = sr[k, pl.ds(wm, 16)]
                er = dr[k, :]
                t = el + er
                s = jnp.exp(jnp.maximum(t, NEG_SLOPE * t))
                sr[k, pl.ds(wm, 16)] = s
                for j in range(nf):
                    if expand_heads:
                        ex = jnp.take_along_axis(s, div8 + (2 * j), axis=0)
                    else:
                        ex = s
                    sr[k, pl.ds(16 * j, 16)] = (
                        sr[k, pl.ds(16 * j, 16)] * ex
                    )
                return carry2

            lax.fori_loop(0, k_ch, edge, 0, unroll=8)

        def do_chunk(cc, u, maybe_first):
            b = u % 3
            pb = (u + 2) % 3
            gwait(b)
            compute(b)
            if maybe_first:
                @pl.when(cc >= 1)
                def _():
                    swait(pb)
            else:
                swait(pb)
            if isinstance(cc, int):
                if cc + 2 < nch:
                    gfire(cc + 2, pb)
            else:
                @pl.when(cc + 2 < nch)
                def _():
                    gfire(jnp.minimum(cc + 2, nch - 1), pb)
            sfire(cc, b)

        gfire(0, 0)
        if nch > 1:
            gfire(1, 1)

        def triple(i, carry):
            c0 = 3 * i
            do_chunk(c0, 0, True)
            do_chunk(c0 + 1, 1, False)
            do_chunk(c0 + 2, 2, False)
            return carry

        lax.fori_loop(0, nt, triple, 0)
        for r in range(rem):
            do_chunk(3 * nt + r, r, 3 * nt + r == 0)
        swait((nch - 1) % 3)
        plsc.subcore_barrier()

        def write_rows(r0):
            pltpu.sync_copy(acc.at[pl.ds(r0, rowb)],
                            comb_h.at[cid, pl.ds(r0, rowb)])

        row_blocks(write_rows)

    return kfn(srctab, dsttab, sidx3, didx3, z)


def kernel(x, edge_index, W1, al1, ar1, W2, al2, ar2):
    n, d_in = x.shape
    h1, f1 = al1.shape
    h2, f2 = al2.shape
    wm1 = h1 * f1
    wm2 = h2 * f2

    idx = jnp.arange(wm1)
    a_l1 = jnp.zeros((wm1, h1), jnp.float32).at[idx, idx // f1].set(
        al1.reshape(wm1))
    a_r1 = jnp.zeros((wm1, h1), jnp.float32).at[idx, idx // f1].set(
        ar1.reshape(wm1))
    ws1 = jnp.concatenate(
        [W1, W1 @ a_l1, jnp.zeros((d_in, 16 - h1), jnp.float32)], axis=1)
    bs1 = jnp.concatenate(
        [jnp.zeros((wm1 + h1,), jnp.float32),
         jnp.full((16 - h1,), PAD, jnp.float32)])[None, :]
    wd1 = jnp.concatenate(
        [W1 @ a_r1, jnp.zeros((d_in, 16 - h1), jnp.float32)], axis=1)
    bd1 = jnp.concatenate(
        [jnp.zeros((h1,), jnp.float32),
         jnp.full((16 - h1,), PAD, jnp.float32)])[None, :]

    eexp = jnp.zeros((16, wm1), jnp.float32).at[idx // f1, idx].set(1.0)

    w_el2 = W2 @ al2.reshape(wm2)
    w_er2 = W2 @ ar2.reshape(wm2)
    ws2 = jnp.concatenate([W2, jnp.tile(w_el2[:, None], (1, 16))], axis=1)
    wd2 = jnp.tile(w_er2[:, None], (1, 16))

    sidx = edge_index[0]
    didx = edge_index[1]
    z1 = jnp.zeros((n, wm1 + 16), jnp.float32)
    z2 = jnp.zeros((n, wm2 + 16), jnp.float32)

    srctab1 = _dense(x, ws1, bs1)
    dsttab1 = _dense(x, wd1, bd1)
    comb1 = _edge_pass(srctab1, dsttab1, sidx, didx, z1,
                       wm=wm1, expand_heads=True)

    srctab2, dsttab2 = _mid(comb1, eexp, ws2, wd2)

    comb2 = _edge_pass(srctab2, dsttab2, sidx, didx, z2,
                       wm=wm2, expand_heads=False)

    return _final(comb2)


# speedup vs baseline: 1.0259x; 1.0202x over previous
import functools

import jax
import jax.numpy as jnp
from jax import lax
from jax.experimental import pallas as pl
from jax.experimental.pallas import tpu as pltpu
from jax.experimental.pallas import tpu_sc as plsc

NEG_SLOPE = 0.2
PAD = -1e30
EPS = 1e-9


def _dense2_body(x_ref, ws_ref, bs_ref, wd_ref, bd_ref, s_ref, d_ref):
    x = x_ref[...]
    s_ref[...] = (
        jnp.dot(x, ws_ref[...], preferred_element_type=jnp.float32)
        + bs_ref[...]
    )
    d_ref[...] = (
        jnp.dot(x, wd_ref[...], preferred_element_type=jnp.float32)
        + bd_ref[...]
    )


def _dense2(x, ws, bs, wd, bd, br=1000):
    n, d = x.shape
    wos = ws.shape[1]
    wod = wd.shape[1]
    return pl.pallas_call(
        _dense2_body,
        grid=(n // br,),
        in_specs=[
            pl.BlockSpec((br, d), lambda i: (i, 0)),
            pl.BlockSpec((d, wos), lambda i: (0, 0)),
            pl.BlockSpec((1, wos), lambda i: (0, 0)),
            pl.BlockSpec((d, wod), lambda i: (0, 0)),
            pl.BlockSpec((1, wod), lambda i: (0, 0)),
        ],
        out_specs=(
            pl.BlockSpec((br, wos), lambda i: (i, 0)),
            pl.BlockSpec((br, wod), lambda i: (i, 0)),
        ),
        out_shape=(
            jax.ShapeDtypeStruct((n, wos), jnp.float32),
            jax.ShapeDtypeStruct((n, wod), jnp.float32),
        ),
    )(x, ws, bs, wd, bd)


def _mid_body(cp_ref, eexp_ref, ws2_ref, wd2_ref, s2_ref, d2_ref, *, wm):
    u = cp_ref[0, :, :wm] + cp_ref[1, :, :wm]
    d = cp_ref[0, :, wm:] + cp_ref[1, :, wm:]
    dexp = jnp.dot(d, eexp_ref[...], preferred_element_type=jnp.float32)
    z = u / (dexp + EPS)
    h = jnp.where(z > 0, z, jnp.exp(z) - 1.0)
    s2_ref[...] = jnp.dot(h, ws2_ref[...], preferred_element_type=jnp.float32)
    d2_ref[...] = jnp.dot(h, wd2_ref[...], preferred_element_type=jnp.float32)


def _mid(comb, eexp, ws2, wd2, br=1000):
    n = comb.shape[1]
    ws = comb.shape[2]
    wm = ws - 16
    w2s = ws2.shape[1]
    w2d = wd2.shape[1]
    return pl.pallas_call(
        functools.partial(_mid_body, wm=wm),
        grid=(n // br,),
        in_specs=[
            pl.BlockSpec((2, br, ws), lambda i: (0, i, 0)),
            pl.BlockSpec((16, wm), lambda i: (0, 0)),
            pl.BlockSpec((wm, w2s), lambda i: (0, 0)),
            pl.BlockSpec((wm, w2d), lambda i: (0, 0)),
        ],
        out_specs=(
            pl.BlockSpec((br, w2s), lambda i: (i, 0)),
            pl.BlockSpec((br, w2d), lambda i: (i, 0)),
        ),
        out_shape=(
            jax.ShapeDtypeStruct((n, w2s), jnp.float32),
            jax.ShapeDtypeStruct((n, w2d), jnp.float32),
        ),
    )(comb, eexp, ws2, wd2)


def _final_body(cp_ref, o_ref, *, wm):
    u = cp_ref[0, :, :wm] + cp_ref[1, :, :wm]
    d = cp_ref[0, :, wm:] + cp_ref[1, :, wm:]
    o_ref[...] = u / (d + EPS)


def _final(comb, br=1000):
    n = comb.shape[1]
    ws = comb.shape[2]
    wm = ws - 16
    return pl.pallas_call(
        functools.partial(_final_body, wm=wm),
        grid=(n // br,),
        in_specs=[
            pl.BlockSpec((2, br, ws), lambda i: (0, i, 0)),
        ],
        out_specs=pl.BlockSpec((br, wm), lambda i: (i, 0)),
        out_shape=jax.ShapeDtypeStruct((n, wm), jnp.float32),
    )(comb)


def _edge_pass(srctab, dsttab, sidx, didx, z, *, wm, expand_heads):
    n = srctab.shape[0]
    e = sidx.shape[0]
    ws = wm + 16
    nc, ns = 2, 16
    nw = nc * ns
    ep = e // nw
    k_ch = min(200, ep)
    nch = ep // k_ch
    nt = nch // 3
    rem = nch % 3
    rowb = 400
    nrb = n // rowb
    nf = wm // 16

    sidx3 = sidx.reshape(nw, nch, k_ch)
    didx3 = didx.reshape(nw, nch, k_ch)

    mesh = plsc.VectorSubcoreMesh(
        core_axis_name="c", subcore_axis_name="s",
        num_cores=nc, num_subcores=ns)

    @functools.partial(
        pl.kernel,
        compiler_params=pltpu.CompilerParams(
            use_tc_tiling_on_sc=False, needs_layout_passes=False),
        out_type=jax.ShapeDtypeStruct((nc, n, ws), jnp.float32),
        mesh=mesh,
        scratch_types=[
            pltpu.VMEM((nch, k_ch), jnp.int32),
            pltpu.VMEM((nch, k_ch), jnp.int32),
            pltpu.VMEM((k_ch, ws), jnp.float32),
            pltpu.VMEM((k_ch, ws), jnp.float32),
            pltpu.VMEM((k_ch, ws), jnp.float32),
            pltpu.VMEM((k_ch, 16), jnp.float32),
            pltpu.VMEM((k_ch, 16), jnp.float32),
            pltpu.VMEM((k_ch, 16), jnp.float32),
            pltpu.VMEM_SHARED((n, ws), jnp.float32),
            pltpu.SemaphoreType.DMA,
            pltpu.SemaphoreType.DMA,
            pltpu.SemaphoreType.DMA,
            pltpu.SemaphoreType.DMA,
            pltpu.SemaphoreType.DMA,
            pltpu.SemaphoreType.DMA,
            pltpu.SemaphoreType.DMA,
            pltpu.SemaphoreType.DMA,
            pltpu.SemaphoreType.DMA,
        ],
    )
    def kfn(srct_h, dstt_h, sidx_h, didx_h, z_h, comb_h,
            sidx_b, didx_b, srows0, srows1, srows2, drows0, drows1, drows2,
            acc, sem_s0, sem_s1, sem_s2, sem_d0, sem_d1, sem_d2,
            sem_c0, sem_c1, sem_c2):
        cid = lax.axis_index("c")
        sid = lax.axis_index("s")
        wid = sid * nc + cid
        srows = (srows0, srows1, srows2)
        drows = (drows0, drows1, drows2)
        sem_s = (sem_s0, sem_s1, sem_s2)
        sem_d = (sem_d0, sem_d1, sem_d2)
        sem_c = (sem_c0, sem_c1, sem_c2)

        def row_blocks(fn):
            for it in range((nrb + ns - 1) // ns):
                rb = sid + it * ns
                if (it + 1) * ns <= nrb:
                    fn(pl.multiple_of(rb * rowb, 8))
                else:
                    @pl.when(rb < nrb)
                    def _():
                        fn(pl.multiple_of(rb * rowb, 8))

        def init_rows(r0):
            pltpu.sync_copy(z_h.at[pl.ds(r0, rowb)], acc.at[pl.ds(r0, rowb)])

        pltpu.sync_copy(sidx_h.at[wid], sidx_b)
        pltpu.sync_copy(didx_h.at[wid], didx_b)
        div8 = lax.iota(jnp.int32, 16) >> 3

        def gfire(cc, b):
            pltpu.async_copy(srct_h.at[sidx_b.at[cc]], srows[b], sem_s[b])
            pltpu.async_copy(dstt_h.at[didx_b.at[cc]], drows[b], sem_d[b])

        def gwait(b):
            pltpu.make_async_copy(
                srct_h.at[sidx_b.at[0]], srows[b], sem_s[b]).wait()
            pltpu.make_async_copy(
                dstt_h.at[didx_b.at[0]], drows[b], sem_d[b]).wait()

        def sfire(cc, b):
            pltpu.async_copy(
                srows[b], acc.at[didx_b.at[cc]], sem_c[b], add=True)

        def swait(b):
            pltpu.make_async_copy(
                srows[b], acc.at[didx_b.at[0]], sem_c[b]).wait()

        def compute(b):
            sr = srows[b]
            dr = drows[b]

            def edge(k, carry2):
                el = sr[k, pl.ds(wm, 16)]
                er = dr[k, :]
                t = el + er
                s = jnp.exp(jnp.maximum(t, NEG_SLOPE * t))
                sr[k, pl.ds(wm, 16)] = s
                for j in range(nf):
                    if expand_heads:
                        ex = jnp.take_along_axis(s, div8 + (2 * j), axis=0)
                    else:
                        ex = s
                    sr[k, pl.ds(16 * j, 16)] = (
                        sr[k, pl.ds(16 * j, 16)] * ex
                    )
                return carry2

            lax.fori_loop(0, k_ch, edge, 0, unroll=8)

        def do_chunk(cc, u, maybe_first):
            b = u % 3
            pb = (u + 2) % 3
            gwait(b)
            compute(b)
            if maybe_first:
                @pl.when(cc >= 1)
                def _():
                    swait(pb)
            else:
                swait(pb)
            if isinstance(cc, int):
                if cc + 2 < nch:
                    gfire(cc + 2, pb)
            else:
                @pl.when(cc + 2 < nch)
                def _():
                    gfire(jnp.minimum(cc + 2, nch - 1), pb)
            sfire(cc, b)

        gfire(0, 0)
        if nch > 1:
            gfire(1, 1)
        row_blocks(init_rows)
        plsc.subcore_barrier()

        def triple(i, carry):
            c0 = 3 * i
            do_chunk(c0, 0, True)
            do_chunk(c0 + 1, 1, False)
            do_chunk(c0 + 2, 2, False)
            return carry

        lax.fori_loop(0, nt, triple, 0)
        for r in range(rem):
            do_chunk(3 * nt + r, r, 3 * nt + r == 0)
        swait((nch - 1) % 3)
        plsc.subcore_barrier()

        def write_rows(r0):
            pltpu.sync_copy(acc.at[pl.ds(r0, rowb)],
                            comb_h.at[cid, pl.ds(r0, rowb)])

        row_blocks(write_rows)

    return kfn(srctab, dsttab, sidx3, didx3, z)


def kernel(x, edge_index, W1, al1, ar1, W2, al2, ar2):
    n, d_in = x.shape
    h1, f1 = al1.shape
    h2, f2 = al2.shape
    wm1 = h1 * f1
    wm2 = h2 * f2

    idx = jnp.arange(wm1)
    a_l1 = jnp.zeros((wm1, h1), jnp.float32).at[idx, idx // f1].set(
        al1.reshape(wm1))
    a_r1 = jnp.zeros((wm1, h1), jnp.float32).at[idx, idx // f1].set(
        ar1.reshape(wm1))
    ws1 = jnp.concatenate(
        [W1, W1 @ a_l1, jnp.zeros((d_in, 16 - h1), jnp.float32)], axis=1)
    bs1 = jnp.concatenate(
        [jnp.zeros((wm1 + h1,), jnp.float32),
         jnp.full((16 - h1,), PAD, jnp.float32)])[None, :]
    wd1 = jnp.concatenate(
        [W1 @ a_r1, jnp.zeros((d_in, 16 - h1), jnp.float32)], axis=1)
    bd1 = jnp.concatenate(
        [jnp.zeros((h1,), jnp.float32),
         jnp.full((16 - h1,), PAD, jnp.float32)])[None, :]

    eexp = jnp.zeros((16, wm1), jnp.float32).at[idx // f1, idx].set(1.0)

    w_el2 = W2 @ al2.reshape(wm2)
    w_er2 = W2 @ ar2.reshape(wm2)
    ws2 = jnp.concatenate([W2, jnp.tile(w_el2[:, None], (1, 16))], axis=1)
    wd2 = jnp.tile(w_er2[:, None], (1, 16))

    sidx = edge_index[0]
    didx = edge_index[1]
    z1 = jnp.zeros((n, wm1 + 16), jnp.float32)
    z2 = jnp.zeros((n, wm2 + 16), jnp.float32)

    srctab1, dsttab1 = _dense2(x, ws1, bs1, wd1, bd1)
    comb1 = _edge_pass(srctab1, dsttab1, sidx, didx, z1,
                       wm=wm1, expand_heads=True)

    srctab2, dsttab2 = _mid(comb1, eexp, ws2, wd2)

    comb2 = _edge_pass(srctab2, dsttab2, sidx, didx, z2,
                       wm=wm2, expand_heads=False)

    return _final(comb2)


# speedup vs baseline: 1.0309x; 1.0049x over previous
import functools

import jax
import jax.numpy as jnp
from jax import lax
from jax.experimental import pallas as pl
from jax.experimental.pallas import tpu as pltpu
from jax.experimental.pallas import tpu_sc as plsc

NEG_SLOPE = 0.2
PAD = -1e30
EPS = 1e-9


def _dense2_body(x_ref, ws_ref, bs_ref, wd_ref, bd_ref, s_ref, d_ref):
    x = x_ref[...]
    s_ref[...] = (
        jnp.dot(x, ws_ref[...], preferred_element_type=jnp.float32)
        + bs_ref[...]
    )
    d_ref[...] = (
        jnp.dot(x, wd_ref[...], preferred_element_type=jnp.float32)
        + bd_ref[...]
    )


def _dense2(x, ws, bs, wd, bd, br=1000):
    n, d = x.shape
    wos = ws.shape[1]
    wod = wd.shape[1]
    return pl.pallas_call(
        _dense2_body,
        grid=(n // br,),
        in_specs=[
            pl.BlockSpec((br, d), lambda i: (i, 0)),
            pl.BlockSpec((d, wos), lambda i: (0, 0)),
            pl.BlockSpec((1, wos), lambda i: (0, 0)),
            pl.BlockSpec((d, wod), lambda i: (0, 0)),
            pl.BlockSpec((1, wod), lambda i: (0, 0)),
        ],
        out_specs=(
            pl.BlockSpec((br, wos), lambda i: (i, 0)),
            pl.BlockSpec((br, wod), lambda i: (i, 0)),
        ),
        out_shape=(
            jax.ShapeDtypeStruct((n, wos), jnp.float32),
            jax.ShapeDtypeStruct((n, wod), jnp.float32),
        ),
    )(x, ws, bs, wd, bd)


def _mid_body(cp_ref, eexp_ref, ws2_ref, wd2_ref, s2_ref, d2_ref, *, wm):
    u = cp_ref[0, :, :wm] + cp_ref[1, :, :wm]
    d = cp_ref[0, :, wm:] + cp_ref[1, :, wm:]
    dexp = jnp.dot(d, eexp_ref[...], preferred_element_type=jnp.float32)
    z = u / (dexp + EPS)
    h = jnp.where(z > 0, z, jnp.exp(z) - 1.0)
    s2_ref[...] = jnp.dot(h, ws2_ref[...], preferred_element_type=jnp.float32)
    d2_ref[...] = jnp.dot(h, wd2_ref[...], preferred_element_type=jnp.float32)


def _mid(comb, eexp, ws2, wd2, br=1000):
    n = comb.shape[1]
    ws = comb.shape[2]
    wm = ws - 16
    w2s = ws2.shape[1]
    w2d = wd2.shape[1]
    return pl.pallas_call(
        functools.partial(_mid_body, wm=wm),
        grid=(n // br,),
        in_specs=[
            pl.BlockSpec((2, br, ws), lambda i: (0, i, 0)),
            pl.BlockSpec((16, wm), lambda i: (0, 0)),
            pl.BlockSpec((wm, w2s), lambda i: (0, 0)),
            pl.BlockSpec((wm, w2d), lambda i: (0, 0)),
        ],
        out_specs=(
            pl.BlockSpec((br, w2s), lambda i: (i, 0)),
            pl.BlockSpec((br, w2d), lambda i: (i, 0)),
        ),
        out_shape=(
            jax.ShapeDtypeStruct((n, w2s), jnp.float32),
            jax.ShapeDtypeStruct((n, w2d), jnp.float32),
        ),
    )(comb, eexp, ws2, wd2)


def _final_body(cp_ref, o_ref, *, wm):
    u = cp_ref[0, :, :wm] + cp_ref[1, :, :wm]
    d = cp_ref[0, :, wm:] + cp_ref[1, :, wm:]
    o_ref[...] = u / (d + EPS)


def _final(comb, br=1000):
    n = comb.shape[1]
    ws = comb.shape[2]
    wm = ws - 16
    return pl.pallas_call(
        functools.partial(_final_body, wm=wm),
        grid=(n // br,),
        in_specs=[
            pl.BlockSpec((2, br, ws), lambda i: (0, i, 0)),
        ],
        out_specs=pl.BlockSpec((br, wm), lambda i: (i, 0)),
        out_shape=jax.ShapeDtypeStruct((n, wm), jnp.float32),
    )(comb)


def _edge_pass(srctab, dsttab, sidx, didx, z, *, wm, expand_heads):
    n = srctab.shape[0]
    e = sidx.shape[0]
    ws = wm + 16
    nc, ns = 2, 16
    nw = nc * ns
    ep = e // nw
    k_ch = min(200, ep)
    nch = ep // k_ch
    nt = nch // 3
    rem = nch % 3
    rowb = 400
    nrb = n // rowb
    nf = wm // 16

    sidx3 = sidx.reshape(nw, nch, k_ch)
    didx3 = didx.reshape(nw, nch, k_ch)

    mesh = plsc.VectorSubcoreMesh(
        core_axis_name="c", subcore_axis_name="s",
        num_cores=nc, num_subcores=ns)

    @functools.partial(
        pl.kernel,
        compiler_params=pltpu.CompilerParams(
            use_tc_tiling_on_sc=False, needs_layout_passes=False),
        out_type=jax.ShapeDtypeStruct((nc, n, ws), jnp.float32),
        mesh=mesh,
        scratch_types=[
            pltpu.VMEM((nch, k_ch), jnp.int32),
            pltpu.VMEM((nch, k_ch), jnp.int32),
            pltpu.VMEM((k_ch, ws), jnp.float32),
            pltpu.VMEM((k_ch, ws), jnp.float32),
            pltpu.VMEM((k_ch, ws), jnp.float32),
            pltpu.VMEM((k_ch, 16), jnp.float32),
            pltpu.VMEM((k_ch, 16), jnp.float32),
            pltpu.VMEM((k_ch, 16), jnp.float32),
            pltpu.VMEM_SHARED((n, ws), jnp.float32),
            pltpu.SemaphoreType.DMA,
            pltpu.SemaphoreType.DMA,
            pltpu.SemaphoreType.DMA,
            pltpu.SemaphoreType.DMA,
            pltpu.SemaphoreType.DMA,
            pltpu.SemaphoreType.DMA,
            pltpu.SemaphoreType.DMA,
            pltpu.SemaphoreType.DMA,
            pltpu.SemaphoreType.DMA,
        ],
    )
    def kfn(srct_h, dstt_h, sidx_h, didx_h, z_h, comb_h,
            sidx_b, didx_b, srows0, srows1, srows2, drows0, drows1, drows2,
            acc, sem_s0, sem_s1, sem_s2, sem_d0, sem_d1, sem_d2,
            sem_c0, sem_c1, sem_c2):
        cid = lax.axis_index("c")
        sid = lax.axis_index("s")
        wid = sid * nc + cid
        srows = (srows0, srows1, srows2)
        drows = (drows0, drows1, drows2)
        sem_s = (sem_s0, sem_s1, sem_s2)
        sem_d = (sem_d0, sem_d1, sem_d2)
        sem_c = (sem_c0, sem_c1, sem_c2)

        def row_blocks(fn):
            for it in range((nrb + ns - 1) // ns):
                rb = sid + it * ns
                if (it + 1) * ns <= nrb:
                    fn(pl.multiple_of(rb * rowb, 8))
                else:
                    @pl.when(rb < nrb)
                    def _():
                        fn(pl.multiple_of(rb * rowb, 8))

        def init_rows(r0):
            pltpu.sync_copy(z_h.at[pl.ds(r0, rowb)], acc.at[pl.ds(r0, rowb)])

        pltpu.sync_copy(sidx_h.at[wid], sidx_b)
        pltpu.sync_copy(didx_h.at[wid], didx_b)
        mod8 = lax.iota(jnp.int32, 16) & 7

        def gfire(cc, b):
            pltpu.async_copy(srct_h.at[sidx_b.at[cc]], srows[b], sem_s[b])
            pltpu.async_copy(dstt_h.at[didx_b.at[cc]], drows[b], sem_d[b])

        def gwait(b):
            pltpu.make_async_copy(
                srct_h.at[sidx_b.at[0]], srows[b], sem_s[b]).wait()
            pltpu.make_async_copy(
                dstt_h.at[didx_b.at[0]], drows[b], sem_d[b]).wait()

        def sfire(cc, b):
            pltpu.async_copy(
                srows[b], acc.at[didx_b.at[cc]], sem_c[b], add=True)

        def swait(b):
            pltpu.make_async_copy(
                srows[b], acc.at[didx_b.at[0]], sem_c[b]).wait()

        def compute(b):
            sr = srows[b]
            dr = drows[b]

            def edge(k, carry2):
                el = sr[k, pl.ds(wm, 16)]
                er = dr[k, :]
                t = el + er
                s = jnp.exp(jnp.maximum(t, NEG_SLOPE * t))
                sr[k, pl.ds(wm, 16)] = s
                if expand_heads:
                    ex = jnp.take_along_axis(s, mod8, axis=0)
                else:
                    ex = s
                for j in range(nf):
                    sr[k, pl.ds(16 * j, 16)] = (
                        sr[k, pl.ds(16 * j, 16)] * ex
                    )
                return carry2

            lax.fori_loop(0, k_ch, edge, 0, unroll=8)

        def do_chunk(cc, u, maybe_first):
            b = u % 3
            pb = (u + 2) % 3
            gwait(b)
            compute(b)
            if maybe_first:
                @pl.when(cc >= 1)
                def _():
                    swait(pb)
            else:
                swait(pb)
            if isinstance(cc, int):
                if cc + 2 < nch:
                    gfire(cc + 2, pb)
            else:
                @pl.when(cc + 2 < nch)
                def _():
                    gfire(jnp.minimum(cc + 2, nch - 1), pb)
            sfire(cc, b)

        gfire(0, 0)
        if nch > 1:
            gfire(1, 1)
        row_blocks(init_rows)
        plsc.subcore_barrier()

        def triple(i, carry):
            c0 = 3 * i
            do_chunk(c0, 0, True)
            do_chunk(c0 + 1, 1, False)
            do_chunk(c0 + 2, 2, False)
            return carry

        lax.fori_loop(0, nt, triple, 0)
        for r in range(rem):
            do_chunk(3 * nt + r, r, 3 * nt + r == 0)
        swait((nch - 1) % 3)
        plsc.subcore_barrier()

        def write_rows(r0):
            pltpu.sync_copy(acc.at[pl.ds(r0, rowb)],
                            comb_h.at[cid, pl.ds(r0, rowb)])

        row_blocks(write_rows)

    return kfn(srctab, dsttab, sidx3, didx3, z)


def kernel(x, edge_index, W1, al1, ar1, W2, al2, ar2):
    n, d_in = x.shape
    h1, f1 = al1.shape
    h2, f2 = al2.shape
    wm1 = h1 * f1
    wm2 = h2 * f2

    idx = jnp.arange(wm1)
    a_l1 = jnp.zeros((wm1, h1), jnp.float32).at[idx, idx // f1].set(
        al1.reshape(wm1))
    a_r1 = jnp.zeros((wm1, h1), jnp.float32).at[idx, idx // f1].set(
        ar1.reshape(wm1))
    perm = (idx % h1) * f1 + idx // h1
    ws1 = jnp.concatenate(
        [W1[:, perm], W1 @ a_l1, jnp.zeros((d_in, 16 - h1), jnp.float32)],
        axis=1)
    bs1 = jnp.concatenate(
        [jnp.zeros((wm1 + h1,), jnp.float32),
         jnp.full((16 - h1,), PAD, jnp.float32)])[None, :]
    wd1 = jnp.concatenate(
        [W1 @ a_r1, jnp.zeros((d_in, 16 - h1), jnp.float32)], axis=1)
    bd1 = jnp.concatenate(
        [jnp.zeros((h1,), jnp.float32),
         jnp.full((16 - h1,), PAD, jnp.float32)])[None, :]

    eexp = jnp.zeros((16, wm1), jnp.float32).at[idx % h1, idx].set(1.0)

    w_el2 = W2 @ al2.reshape(wm2)
    w_er2 = W2 @ ar2.reshape(wm2)
    ws2 = jnp.concatenate(
        [W2[perm], jnp.tile(w_el2[perm][:, None], (1, 16))], axis=1)
    wd2 = jnp.tile(w_er2[perm][:, None], (1, 16))

    sidx = edge_index[0]
    didx = edge_index[1]
    z1 = jnp.zeros((n, wm1 + 16), jnp.float32)
    z2 = jnp.zeros((n, wm2 + 16), jnp.float32)

    srctab1, dsttab1 = _dense2(x, ws1, bs1, wd1, bd1)
    comb1 = _edge_pass(srctab1, dsttab1, sidx, didx, z1,
                       wm=wm1, expand_heads=True)

    srctab2, dsttab2 = _mid(comb1, eexp, ws2, wd2)

    comb2 = _edge_pass(srctab2, dsttab2, sidx, didx, z2,
                       wm=wm2, expand_heads=False)

    return _final(comb2)
